# baseline (speedup 1.0000x reference)
import functools

import jax
import jax.numpy as jnp
from jax import lax
from jax.experimental import pallas as pl
from jax.experimental.pallas import tpu as pltpu
from jax.experimental.pallas import tpu_sc as plsc

N_NODES = 50000
EMBED_DIM = 64
NUM_CORES = 2
NUM_SUBCORES = 16
NUM_WORKERS = NUM_CORES * NUM_SUBCORES
CHUNK = 400
NUM_CHUNKS = N_NODES // CHUNK

_mesh = plsc.VectorSubcoreMesh(core_axis_name="c", subcore_axis_name="s")


@functools.partial(
    pl.kernel,
    mesh=_mesh,
    compiler_params=pltpu.CompilerParams(use_tc_tiling_on_sc=False),
    out_type=jax.ShapeDtypeStruct((N_NODES, EMBED_DIM + 1), jnp.float32),
    scratch_types=[
        pltpu.VMEM((CHUNK,), jnp.int32),
        pltpu.VMEM((CHUNK, EMBED_DIM), jnp.float32),
        pltpu.VMEM((CHUNK, 1), jnp.float32),
        pltpu.SemaphoreType.DMA,
    ],
)
def _gather_concat(table_hbm, idx_hbm, flag_hbm, out_hbm, idx_v, rows_v,
                   flag_v, sem):
    wid = lax.axis_index("s") * NUM_CORES + lax.axis_index("c")

    def do_chunk(c):
        base = c * CHUNK
        pltpu.sync_copy(idx_hbm.at[pl.ds(base, CHUNK)], idx_v)
        pltpu.async_copy(table_hbm.at[idx_v], rows_v, sem).wait()
        pltpu.sync_copy(rows_v, out_hbm.at[pl.ds(base, CHUNK),
                                           pl.ds(0, EMBED_DIM)])
        pltpu.sync_copy(flag_hbm.at[pl.ds(base, CHUNK)], flag_v)
        pltpu.sync_copy(flag_v, out_hbm.at[pl.ds(base, CHUNK),
                                           pl.ds(EMBED_DIM, 1)])

    for k in range(NUM_CHUNKS // NUM_WORKERS):
        do_chunk(wid + k * NUM_WORKERS)

    @pl.when(wid + 3 * NUM_WORKERS < NUM_CHUNKS)
    def _():
        do_chunk(wid + 3 * NUM_WORKERS)


def kernel(table, indices, is_candidate):
    return _gather_concat(table, indices.astype(jnp.int32),
                          is_candidate.reshape(N_NODES, 1))


# speedup vs baseline: 1.5165x; 1.5165x over previous
import functools

import jax
import jax.numpy as jnp
from jax import lax
from jax.experimental import pallas as pl
from jax.experimental.pallas import tpu as pltpu
from jax.experimental.pallas import tpu_sc as plsc

N_NODES = 50000
EMBED_DIM = 64
NUM_CORES = 2
NUM_SUBCORES = 16
NUM_WORKERS = NUM_CORES * NUM_SUBCORES
CHUNK = 80
NUM_CHUNKS = N_NODES // CHUNK
FULL_ROUNDS = NUM_CHUNKS // NUM_WORKERS
GRP = 16

_mesh = plsc.VectorSubcoreMesh(core_axis_name="c", subcore_axis_name="s")


@functools.partial(
    pl.kernel,
    mesh=_mesh,
    out_type=jax.ShapeDtypeStruct((N_NODES, EMBED_DIM + 1), jnp.float32),
    scratch_types=[
        pltpu.VMEM((CHUNK,), jnp.int32),
        pltpu.VMEM((CHUNK, 8, EMBED_DIM), jnp.float32),
        pltpu.VMEM((CHUNK, EMBED_DIM + 1), jnp.float32),
        pltpu.VMEM((CHUNK,), jnp.float32),
        pltpu.SemaphoreType.DMA,
    ],
)
def _gather_concat(table_hbm, idx_hbm, flag_hbm, out_hbm, idx_v, blocks_v,
                   out_v, flag_v, sem):
    wid = lax.axis_index("s") * NUM_CORES + lax.axis_index("c")
    last_lane = lax.iota(jnp.int32, 16) == 15

    def do_chunk(c):
        base = c * CHUNK
        pltpu.sync_copy(idx_hbm.at[pl.ds(base, CHUNK)], idx_v)
        pltpu.sync_copy(flag_hbm.at[pl.ds(base, CHUNK)], flag_v)

        def grp_body(g, carry):
            ivec = idx_v[pl.ds(g * GRP, GRP)]
            fvec = flag_v[pl.ds(g * GRP, GRP)]
            bvec = lax.shift_right_logical(ivec, 3)
            svec = lax.bitwise_and(ivec, 7)
            copies = []
            for t in range(GRP):
                r = g * GRP + t
                copies.append(pltpu.async_copy(
                    table_hbm.at[bvec[t]], blocks_v.at[r], sem))
            for t in range(GRP):
                copies[t].wait()
            for t in range(GRP):
                r = g * GRP + t
                for k in range(EMBED_DIM // 16):
                    out_v[r, pl.ds(k * 16, 16)] = (
                        blocks_v[r, svec[t], pl.ds(k * 16, 16)])
                cur = out_v[r, pl.ds(EMBED_DIM - 15, 16)]
                out_v[r, pl.ds(EMBED_DIM - 15, 16)] = jnp.where(
                    last_lane, lax.broadcast(fvec[t], (16,)), cur)
            return carry

        lax.fori_loop(0, CHUNK // GRP, grp_body, 0)
        pltpu.sync_copy(out_v, out_hbm.at[pl.ds(base, CHUNK)])

    for k in range(FULL_ROUNDS):
        do_chunk(wid + k * NUM_WORKERS)

    @pl.when(wid + FULL_ROUNDS * NUM_WORKERS < NUM_CHUNKS)
    def _():
        do_chunk(wid + FULL_ROUNDS * NUM_WORKERS)


def kernel(table, indices, is_candidate):
    table3 = table.reshape(125000, 8, EMBED_DIM)
    return _gather_concat(table3, indices.astype(jnp.int32), is_candidate)
